# baseline (speedup 1.0000x reference)
import functools

import jax
import jax.numpy as jnp
from jax import lax
from jax.experimental import pallas as pl
from jax.experimental.pallas import tpu as pltpu
from jax.experimental.pallas import tpu_sc as plsc

_MOMENTUM = 0.9
_B, _N, _D, _M = 1024, 77, 768, 100000
_NC, _NS, _L = 2, 16, 16
_NW = _NC * _NS
_BPW = _B // _NW

_mesh = plsc.VectorSubcoreMesh(
    core_axis_name="c", subcore_axis_name="s", num_cores=_NC, num_subcores=_NS
)


@functools.partial(
    pl.kernel,
    out_type=(),
    mesh=_mesh,
    scratch_types=[
        pltpu.VMEM((_BPW,), jnp.int32),
        pltpu.VMEM((_BPW,), jnp.int32),
        pltpu.VMEM((_BPW,), jnp.int32),
        pltpu.VMEM((_BPW, _D), jnp.float32),
        pltpu.VMEM((_BPW, _D), jnp.float32),
        pltpu.SemaphoreType.DMA,
    ],
)
def _sc_update(meta_hbm, ids_hbm, pos_hbm, emb_ref, mem_ref,
               idx_v, pos_v, eidx_v, rows_v, meta_v, sem):
    wid = lax.axis_index("s") * _NC + lax.axis_index("c")
    base = wid * _BPW

    pltpu.sync_copy(ids_hbm.at[pl.ds(base, _BPW)], idx_v)
    pltpu.sync_copy(pos_hbm.at[pl.ds(base, _BPW)], pos_v)
    pltpu.sync_copy(meta_hbm.at[pl.ds(base, _BPW)], meta_v)

    pltpu.async_copy(mem_ref.at[idx_v], rows_v, sem).wait()

    plsc.subcore_barrier()

    def _row(r, carry):
        for c in range(_D // _L):
            s = pl.ds(c * _L, _L)
            rows_v[r, s] = (
                rows_v[r, s] * _MOMENTUM + meta_v[r, s] * (1.0 - _MOMENTUM)
            )
        return carry

    lax.fori_loop(0, _BPW, _row, 0)

    for c in range(_BPW // _L):
        s = pl.ds(c * _L, _L)
        row_id = base + c * _L + lax.iota(jnp.int32, _L)
        eidx_v[s] = pos_v[s] * _B + row_id

    upd_mem = pltpu.async_copy(rows_v, mem_ref.at[idx_v], sem)
    upd_emb = pltpu.async_copy(rows_v, emb_ref.at[eidx_v], sem)
    upd_mem.wait()
    upd_emb.wait()


_EPW = (_N * _B) // _NW
_MPW = 3128


@functools.partial(
    pl.kernel,
    out_type=(
        jax.ShapeDtypeStruct((_N * _B, _D), jnp.float32),
        jax.ShapeDtypeStruct((_M, _D), jnp.float32),
    ),
    mesh=_mesh,
    scratch_types=[pltpu.SemaphoreType.DMA, pltpu.SemaphoreType.DMA],
)
def _sc_bulkcopy(emb_hbm, mem_hbm, emb_out, mem_out, sem_a, sem_b):
    wid = lax.axis_index("s") * _NC + lax.axis_index("c")
    e_lo = wid * _EPW
    m_lo = jnp.minimum(wid * _MPW, _M - _MPW)
    cp_e = pltpu.async_copy(
        emb_hbm.at[pl.ds(e_lo, _EPW)], emb_out.at[pl.ds(e_lo, _EPW)], sem_a
    )
    cp_m = pltpu.async_copy(
        mem_hbm.at[pl.ds(m_lo, _MPW)], mem_out.at[pl.ds(m_lo, _MPW)], sem_b
    )
    cp_e.wait()
    cp_m.wait()


def kernel(embedded_text, meta, mem, ids, pos):
    emb_in = jnp.swapaxes(embedded_text, 0, 1).reshape(_N * _B, _D)
    emb_copy, mem_copy = _sc_bulkcopy(emb_in, mem)
    emb_ref = jax.new_ref(emb_copy)
    mem_ref = jax.new_ref(mem_copy)
    _sc_update(meta, ids.astype(jnp.int32), pos.astype(jnp.int32),
               emb_ref, mem_ref)
    new_emb = jnp.swapaxes(jax.freeze(emb_ref).reshape(_N, _B, _D), 0, 1)
    return new_emb, jax.freeze(mem_ref)
